# baseline (speedup 1.0000x reference)
import jax
import jax.numpy as jnp
from jax.experimental import pallas as pl
from jax.experimental.pallas import tpu as pltpu

_M = 4096
_K = 4096
_BI = 32
_BM = 2048
_BN = 256


def _decode_kernel(t5_ref, lut_ref, out_ref):
    e = t5_ref[:, 0].astype(jnp.uint32)
    o = t5_ref[:, 1].astype(jnp.uint32)
    en = jnp.concatenate([e[:, 1:, :], e[:, :1, :]], axis=1)
    on = jnp.concatenate([o[:, 1:, :], o[:, :1, :]], axis=1)
    r = _BI * 16
    e = e.reshape(r, 256)
    o = o.reshape(r, 256)
    en = en.reshape(r, 256)
    on = on.reshape(r, 256)
    u_e = (e << 16) | o
    u_o = (o << 16) | en
    u_n = (en << 16) | on
    codes = (
        (u_e >> 16), (u_e >> 12), (u_e >> 8), (u_o >> 20),
        (u_o >> 16), (u_o >> 12), (u_o >> 8), (u_n >> 20),
    )
    tabs = [
        jnp.broadcast_to(lut_ref[p, :].reshape(1, 128), (r, 128))
        for p in range(4)
    ]
    parts = []
    for c in range(8):
        code = (codes[c] & 511).astype(jnp.int32)
        lo = code & 127
        hi = code >> 7
        m0 = hi == 0
        m1 = hi == 1
        m2 = hi == 2
        halves0, halves1 = [], []
        for h in range(2):
            sl = slice(h * 128, (h + 1) * 128)
            g0 = jnp.take_along_axis(tabs[0], lo[:, sl], axis=-1)
            g1 = jnp.take_along_axis(tabs[1], lo[:, sl], axis=-1)
            g2 = jnp.take_along_axis(tabs[2], lo[:, sl], axis=-1)
            g3 = jnp.take_along_axis(tabs[3], lo[:, sl], axis=-1)
            g = jnp.where(
                m0[:, sl], g0,
                jnp.where(m1[:, sl], g1, jnp.where(m2[:, sl], g2, g3)))
            halves0.append(pltpu.bitcast(g & jnp.int32(-65536), jnp.float32))
            halves1.append(pltpu.bitcast(g << 16, jnp.float32))
        parts.append(jnp.concatenate(halves0, axis=-1))
        parts.append(jnp.concatenate(halves1, axis=-1))
    out = jnp.concatenate(parts, axis=-1)
    out_ref[...] = out.reshape(_BI, 16, _K).astype(jnp.bfloat16)


def _matmul_kernel(x_ref, w_ref, o_ref):
    o_ref[...] = jax.lax.dot_general(
        x_ref[...], w_ref[...],
        (((0,), (1,)), ((), ())),
        preferred_element_type=jnp.float32)


def kernel(inp, trellis, tlut):
    bs = inp.shape[0] * inp.shape[1]
    x = inp.reshape(bs, _K)
    t5 = trellis.reshape(256, 256, 16, 2).transpose(0, 3, 2, 1)
    tb = jax.lax.bitcast_convert_type(
        tlut.astype(jnp.bfloat16), jnp.uint16).astype(jnp.uint32)
    lut4 = jax.lax.bitcast_convert_type(
        (tb[:, 0] << 16) | tb[:, 1], jnp.int32).reshape(4, 128)

    wt = pl.pallas_call(
        _decode_kernel,
        grid=(256 // _BI,),
        in_specs=[
            pl.BlockSpec((_BI, 2, 16, 256), lambda i: (i, 0, 0, 0)),
            pl.BlockSpec((4, 128), lambda i: (0, 0)),
        ],
        out_specs=pl.BlockSpec((_BI, 16, _K), lambda i: (i, 0, 0)),
        out_shape=jax.ShapeDtypeStruct((256, 16, _K), jnp.bfloat16),
        compiler_params=pltpu.CompilerParams(
            dimension_semantics=("parallel",),
        ),
    )(t5, lut4)
    wt = wt.reshape(_M, _K)
    xtt = x.astype(jnp.bfloat16).T
    xtt = xtt.reshape(256, 16, bs).transpose(1, 0, 2).reshape(_K, bs)

    bm = min(_BM, bs)
    y = pl.pallas_call(
        _matmul_kernel,
        grid=(bs // bm, _M // _BN),
        in_specs=[
            pl.BlockSpec((_K, bm), lambda b, m: (0, b)),
            pl.BlockSpec((_BN, _K), lambda b, m: (m, 0)),
        ],
        out_specs=pl.BlockSpec((bm, _BN), lambda b, m: (b, m)),
        out_shape=jax.ShapeDtypeStruct((bs, _M), jnp.float32),
        compiler_params=pltpu.CompilerParams(
            dimension_semantics=("parallel", "arbitrary"),
            vmem_limit_bytes=100 * 1024 * 1024,
        ),
    )(xtt, wt)
    return y.reshape(*inp.shape[:-1], _M).astype(inp.dtype)


# speedup vs baseline: 1.1019x; 1.1019x over previous
import jax
import jax.numpy as jnp
from jax.experimental import pallas as pl
from jax.experimental.pallas import tpu as pltpu

_M = 4096
_K = 4096
_BI = 16
_BM = 2048
_BN = 512


def _decode_kernel(t5_ref, lut_ref, out_ref):
    e = t5_ref[:, 0].astype(jnp.uint32)
    o = t5_ref[:, 1].astype(jnp.uint32)
    en = jnp.concatenate([e[:, 1:, :], e[:, :1, :]], axis=1)
    on = jnp.concatenate([o[:, 1:, :], o[:, :1, :]], axis=1)
    r = _BI * 16
    e = e.reshape(r, 256)
    o = o.reshape(r, 256)
    en = en.reshape(r, 256)
    on = on.reshape(r, 256)
    u_e = (e << 16) | o
    u_o = (o << 16) | en
    u_n = (en << 16) | on
    codes = (
        (u_e >> 16), (u_e >> 12), (u_e >> 8), (u_o >> 20),
        (u_o >> 16), (u_o >> 12), (u_o >> 8), (u_n >> 20),
    )
    tabs = [
        jnp.broadcast_to(lut_ref[p, :].reshape(1, 128), (r, 128))
        for p in range(4)
    ]
    parts = []
    for c in range(8):
        code = (codes[c] & 511).astype(jnp.int32)
        lo = code & 127
        hi = code >> 7
        m0 = hi == 0
        m1 = hi == 1
        m2 = hi == 2
        halves0, halves1 = [], []
        for h in range(2):
            sl = slice(h * 128, (h + 1) * 128)
            g0 = jnp.take_along_axis(tabs[0], lo[:, sl], axis=-1)
            g1 = jnp.take_along_axis(tabs[1], lo[:, sl], axis=-1)
            g2 = jnp.take_along_axis(tabs[2], lo[:, sl], axis=-1)
            g3 = jnp.take_along_axis(tabs[3], lo[:, sl], axis=-1)
            g = jnp.where(
                m0[:, sl], g0,
                jnp.where(m1[:, sl], g1, jnp.where(m2[:, sl], g2, g3)))
            halves0.append(pltpu.bitcast(g & jnp.int32(-65536), jnp.float32))
            halves1.append(pltpu.bitcast(g << 16, jnp.float32))
        parts.append(jnp.concatenate(halves0, axis=-1))
        parts.append(jnp.concatenate(halves1, axis=-1))
    out = jnp.concatenate(parts, axis=-1)
    out_ref[...] = out.reshape(_BI, 16, _K).astype(jnp.bfloat16)


def _matmul_kernel(x_ref, w_ref, o_ref):
    o_ref[...] = jax.lax.dot_general(
        x_ref[...], w_ref[...],
        (((0,), (1,)), ((), ())),
        preferred_element_type=jnp.float32)


def kernel(inp, trellis, tlut):
    bs = inp.shape[0] * inp.shape[1]
    x = inp.reshape(bs, _K)
    t5 = trellis.reshape(256, 256, 16, 2).transpose(0, 3, 2, 1)
    tb = jax.lax.bitcast_convert_type(
        tlut.astype(jnp.bfloat16), jnp.uint16).astype(jnp.uint32)
    lut4 = jax.lax.bitcast_convert_type(
        (tb[:, 0] << 16) | tb[:, 1], jnp.int32).reshape(4, 128)

    wt = pl.pallas_call(
        _decode_kernel,
        grid=(256 // _BI,),
        in_specs=[
            pl.BlockSpec((_BI, 2, 16, 256), lambda i: (i, 0, 0, 0)),
            pl.BlockSpec((4, 128), lambda i: (0, 0)),
        ],
        out_specs=pl.BlockSpec((_BI, 16, _K), lambda i: (i, 0, 0)),
        out_shape=jax.ShapeDtypeStruct((256, 16, _K), jnp.bfloat16),
        compiler_params=pltpu.CompilerParams(
            dimension_semantics=("parallel",),
        ),
    )(t5, lut4)
    wt = wt.reshape(_M, _K)
    xtt = x.astype(jnp.bfloat16).T
    xtt = xtt.reshape(256, 16, bs).transpose(1, 0, 2).reshape(_K, bs)

    bm = min(_BM, bs)
    y = pl.pallas_call(
        _matmul_kernel,
        grid=(bs // bm, _M // _BN),
        in_specs=[
            pl.BlockSpec((_K, bm), lambda b, m: (0, b)),
            pl.BlockSpec((_BN, _K), lambda b, m: (m, 0)),
        ],
        out_specs=pl.BlockSpec((bm, _BN), lambda b, m: (b, m)),
        out_shape=jax.ShapeDtypeStruct((bs, _M), jnp.float32),
        compiler_params=pltpu.CompilerParams(
            dimension_semantics=("parallel", "arbitrary"),
            vmem_limit_bytes=100 * 1024 * 1024,
        ),
    )(xtt, wt)
    return y.reshape(*inp.shape[:-1], _M).astype(inp.dtype)


# speedup vs baseline: 1.1036x; 1.0016x over previous
import jax
import jax.numpy as jnp
from jax.experimental import pallas as pl
from jax.experimental.pallas import tpu as pltpu
from jax.experimental.xla_metadata import set_xla_metadata

_M = 4096
_K = 4096
_BI = 16
_BM = 2048
_BN = 512


def _decode_kernel(t5_ref, lut_ref, out_ref):
    e = t5_ref[:, 0].astype(jnp.uint32)
    o = t5_ref[:, 1].astype(jnp.uint32)
    en = jnp.concatenate([e[:, 1:, :], e[:, :1, :]], axis=1)
    on = jnp.concatenate([o[:, 1:, :], o[:, :1, :]], axis=1)
    r = _BI * 16
    e = e.reshape(r, 256)
    o = o.reshape(r, 256)
    en = en.reshape(r, 256)
    on = on.reshape(r, 256)
    u_e = (e << 16) | o
    u_o = (o << 16) | en
    u_n = (en << 16) | on
    codes = (
        (u_e >> 16), (u_e >> 12), (u_e >> 8), (u_o >> 20),
        (u_o >> 16), (u_o >> 12), (u_o >> 8), (u_n >> 20),
    )
    tabs = [
        jnp.broadcast_to(lut_ref[p, :].reshape(1, 128), (r, 128))
        for p in range(4)
    ]
    parts = []
    for c in range(8):
        code = (codes[c] & 511).astype(jnp.int32)
        lo = code & 127
        hi = code >> 7
        m0 = hi == 0
        m1 = hi == 1
        m2 = hi == 2
        halves0, halves1 = [], []
        for h in range(2):
            sl = slice(h * 128, (h + 1) * 128)
            g0 = jnp.take_along_axis(tabs[0], lo[:, sl], axis=-1)
            g1 = jnp.take_along_axis(tabs[1], lo[:, sl], axis=-1)
            g2 = jnp.take_along_axis(tabs[2], lo[:, sl], axis=-1)
            g3 = jnp.take_along_axis(tabs[3], lo[:, sl], axis=-1)
            g = jnp.where(
                m0[:, sl], g0,
                jnp.where(m1[:, sl], g1, jnp.where(m2[:, sl], g2, g3)))
            halves0.append(pltpu.bitcast(g & jnp.int32(-65536), jnp.float32))
            halves1.append(pltpu.bitcast(g << 16, jnp.float32))
        parts.append(jnp.concatenate(halves0, axis=-1))
        parts.append(jnp.concatenate(halves1, axis=-1))
    out = jnp.concatenate(parts, axis=-1)
    out_ref[...] = out.reshape(_BI, 16, _K).astype(jnp.bfloat16)


def _matmul_kernel(x_ref, w_ref, o_ref):
    o_ref[...] = jax.lax.dot_general(
        x_ref[...], w_ref[...],
        (((0,), (1,)), ((), ())),
        preferred_element_type=jnp.float32)


def kernel(inp, trellis, tlut):
    bs = inp.shape[0] * inp.shape[1]
    x = inp.reshape(bs, _K)
    t5 = trellis.reshape(256, 256, 16, 2).transpose(0, 3, 2, 1)
    tb = jax.lax.bitcast_convert_type(
        tlut.astype(jnp.bfloat16), jnp.uint16).astype(jnp.uint32)
    lut4 = jax.lax.bitcast_convert_type(
        (tb[:, 0] << 16) | tb[:, 1], jnp.int32).reshape(4, 128)

    with set_xla_metadata(_scheduling_group_id=0):
        wt = pl.pallas_call(
            _decode_kernel,
            grid=(256 // _BI,),
            in_specs=[
                pl.BlockSpec((_BI, 2, 16, 256), lambda i: (i, 0, 0, 0)),
                pl.BlockSpec((4, 128), lambda i: (0, 0)),
            ],
            out_specs=pl.BlockSpec((_BI, 16, _K), lambda i: (i, 0, 0)),
            out_shape=jax.ShapeDtypeStruct((256, 16, _K), jnp.bfloat16),
            compiler_params=pltpu.CompilerParams(
                dimension_semantics=("parallel",),
            ),
        )(t5, lut4)
    wt = wt.reshape(_M, _K)
    with set_xla_metadata(_scheduling_group_id=0):
        xtt = x.astype(jnp.bfloat16).T
        xtt = xtt.reshape(256, 16, bs).transpose(1, 0, 2).reshape(_K, bs)

    bm = min(_BM, bs)
    y = pl.pallas_call(
        _matmul_kernel,
        grid=(bs // bm, _M // _BN),
        in_specs=[
            pl.BlockSpec((_K, bm), lambda b, m: (0, b)),
            pl.BlockSpec((_BN, _K), lambda b, m: (m, 0)),
        ],
        out_specs=pl.BlockSpec((bm, _BN), lambda b, m: (b, m)),
        out_shape=jax.ShapeDtypeStruct((bs, _M), jnp.float32),
        compiler_params=pltpu.CompilerParams(
            dimension_semantics=("parallel", "arbitrary"),
            vmem_limit_bytes=100 * 1024 * 1024,
        ),
    )(xtt, wt)
    return y.reshape(*inp.shape[:-1], _M).astype(inp.dtype)


# speedup vs baseline: 1.1038x; 1.0001x over previous
import jax
import jax.numpy as jnp
from jax.experimental import pallas as pl
from jax.experimental.pallas import tpu as pltpu

_M = 4096
_K = 4096
_BI = 16
_BM = 2048
_BN = 512


def _decode_kernel(t5_ref, lut_ref, out_ref):
    e = t5_ref[:, 0].astype(jnp.uint32)
    o = t5_ref[:, 1].astype(jnp.uint32)
    en = jnp.concatenate([e[:, 1:, :], e[:, :1, :]], axis=1)
    on = jnp.concatenate([o[:, 1:, :], o[:, :1, :]], axis=1)
    r = _BI * 16
    e = e.reshape(r, 256)
    o = o.reshape(r, 256)
    en = en.reshape(r, 256)
    on = on.reshape(r, 256)
    u_e = (e << 16) | o
    u_o = (o << 16) | en
    u_n = (en << 16) | on
    codes = (
        (u_e >> 16), (u_e >> 12), (u_e >> 8), (u_o >> 20),
        (u_o >> 16), (u_o >> 12), (u_o >> 8), (u_n >> 20),
    )
    tabs = [
        jnp.broadcast_to(lut_ref[p, :].reshape(1, 128), (r, 128))
        for p in range(4)
    ]
    parts = []
    for c in range(8):
        code = (codes[c] & 511).astype(jnp.int32)
        lo = code & 127
        hi = code >> 7
        m0 = hi == 0
        m1 = hi == 1
        m2 = hi == 2
        halves0, halves1 = [], []
        for h in range(2):
            sl = slice(h * 128, (h + 1) * 128)
            g0 = jnp.take_along_axis(tabs[0], lo[:, sl], axis=-1)
            g1 = jnp.take_along_axis(tabs[1], lo[:, sl], axis=-1)
            g2 = jnp.take_along_axis(tabs[2], lo[:, sl], axis=-1)
            g3 = jnp.take_along_axis(tabs[3], lo[:, sl], axis=-1)
            g = jnp.where(
                m0[:, sl], g0,
                jnp.where(m1[:, sl], g1, jnp.where(m2[:, sl], g2, g3)))
            halves0.append(pltpu.bitcast(g & jnp.int32(-65536), jnp.float32))
            halves1.append(pltpu.bitcast(g << 16, jnp.float32))
        parts.append(jnp.concatenate(halves0, axis=-1))
        parts.append(jnp.concatenate(halves1, axis=-1))
    out = jnp.concatenate(parts, axis=-1)
    out_ref[...] = out.reshape(_BI, 16, _K).astype(jnp.bfloat16)


def _matmul_kernel(x_ref, w_ref, o_ref):
    o_ref[...] = jax.lax.dot_general(
        x_ref[...], w_ref[...],
        (((0,), (1,)), ((), ())),
        preferred_element_type=jnp.float32)


def kernel(inp, trellis, tlut):
    bs = inp.shape[0] * inp.shape[1]
    x = inp.reshape(bs, _K)
    t5 = trellis.reshape(256, 256, 16, 2).transpose(0, 3, 2, 1)
    tb = jax.lax.bitcast_convert_type(
        tlut.astype(jnp.bfloat16), jnp.uint16).astype(jnp.uint32)
    lut4 = jax.lax.bitcast_convert_type(
        (tb[:, 0] << 16) | tb[:, 1], jnp.int32).reshape(4, 128)

    wt = pl.pallas_call(
        _decode_kernel,
        grid=(256 // _BI,),
        in_specs=[
            pl.BlockSpec((_BI, 2, 16, 256), lambda i: (i, 0, 0, 0)),
            pl.BlockSpec((4, 128), lambda i: (0, 0)),
        ],
        out_specs=pl.BlockSpec((_BI, 16, _K), lambda i: (i, 0, 0)),
        out_shape=jax.ShapeDtypeStruct((256, 16, _K), jnp.bfloat16),
        compiler_params=pltpu.CompilerParams(
            dimension_semantics=("parallel",),
        ),
    )(t5, lut4)
    wt = wt.reshape(_M, _K)
    xtt = x.astype(jnp.bfloat16).T
    xtt = xtt.reshape(256, 16, bs).transpose(1, 0, 2).reshape(_K, bs)

    bm = min(_BM, bs)
    y = pl.pallas_call(
        _matmul_kernel,
        grid=(bs // bm, _M // _BN),
        in_specs=[
            pl.BlockSpec((_K, bm), lambda b, m: (0, b)),
            pl.BlockSpec((_BN, _K), lambda b, m: (m, 0)),
        ],
        out_specs=pl.BlockSpec((bm, _BN), lambda b, m: (b, m)),
        out_shape=jax.ShapeDtypeStruct((bs, _M), jnp.float32),
        compiler_params=pltpu.CompilerParams(
            dimension_semantics=("parallel", "arbitrary"),
            vmem_limit_bytes=100 * 1024 * 1024,
        ),
    )(xtt, wt)
    return y.reshape(*inp.shape[:-1], _M).astype(inp.dtype)
